# baseline (speedup 1.0000x reference)
import jax
import jax.numpy as jnp
from jax.experimental import pallas as pl
from jax.experimental.pallas import tpu as pltpu

_B, _N, _FIN, _FOUT, _H, _M = 8, 166, 256, 512, 8, 16
_NH = _FOUT // _H
_SLOPE = 0.2
_NP = 256
_BF = jnp.bfloat16


def _lrelu(x):
    return jnp.where(x > 0, x, _SLOPE * x)


def _trunc_bf(x):
    i = jax.lax.bitcast_convert_type(x, jnp.int32)
    i = jnp.bitwise_and(i, jnp.int32(-65536))
    return jax.lax.bitcast_convert_type(i, jnp.float32).astype(_BF)


def _proj_kernel(h_ref, wl_ref, wr_ref, gl_ref, gr_ref):
    dn = (((1,), (1,)), ((), ()))
    hb = h_ref[...]
    gl_ref[...] = jax.lax.dot_general(hb, wl_ref[...], dn,
                                      preferred_element_type=jnp.float32)
    gr_ref[...] = jax.lax.dot_general(hb, wr_ref[...], dn,
                                      preferred_element_type=jnp.float32)


def _score_kernel(gl_ref, gr_ref, adj_ref, aw_ref, e_ref):
    adjb = adj_ref[0]
    neg = jnp.float32(-jnp.inf)
    wcol = aw_ref[0].astype(_BF).reshape(_NH, 1)
    for sub in range(2):
        gl = gl_ref[0, :, sub * _NH:(sub + 1) * _NH]
        gr = gr_ref[0, :, sub * _NH:(sub + 1) * _NH]
        t = gl[None, :, :] + gr[:, None, :]
        act = _lrelu(t).astype(_BF).reshape(_N * _N, _NH)
        e = jax.lax.dot_general(act, wcol, (((1,), (0,)), ((), ())),
                                preferred_element_type=jnp.float32)
        e = e.reshape(_N, _N)
        e = jnp.where(adjb == 0.0, neg, e)
        e_ref[0, sub] = jnp.concatenate(
            [e, jnp.full((_N, _NP - _N), neg, jnp.float32)], axis=1)


def _select_kernel(e_ref, gr_ref, out_ref):
    neg = jnp.float32(-jnp.inf)
    for sub in range(2):
        e = e_ref[0, sub]
        rowmax = jnp.max(e, axis=1, keepdims=True)
        work = e
        thr = rowmax
        for _ in range(_M):
            thr = jnp.max(work, axis=1, keepdims=True)
            work = jnp.where(work >= thr, neg, work)
        p = jnp.where(e >= thr, jnp.exp(e - rowmax), 0.0)
        s = jnp.sum(p, axis=1, keepdims=True)
        a = (p / s).astype(_BF)
        gr = gr_ref[0, :, sub * _NH:(sub + 1) * _NH]
        grp = jnp.concatenate(
            [gr, jnp.zeros((_NP - _N, _NH), jnp.float32)], axis=0)
        out = jax.lax.dot_general(a, grp.astype(_BF), (((1,), (0,)), ((), ())),
                                  preferred_element_type=jnp.float32)
        out_ref[0, sub] = _lrelu(out)


def kernel(h, adj, W_l, W_r, attn_w, attn_b):
    del attn_b
    h2 = h.reshape(_B * _N, _FIN)
    gl, gr = pl.pallas_call(
        _proj_kernel,
        grid=(1,),
        in_specs=[pl.BlockSpec((_B * _N, _FIN), lambda g: (0, 0)),
                  pl.BlockSpec((_FOUT, _FIN), lambda g: (0, 0)),
                  pl.BlockSpec((_FOUT, _FIN), lambda g: (0, 0))],
        out_specs=[pl.BlockSpec((_B * _N, _FOUT), lambda g: (0, 0)),
                   pl.BlockSpec((_B * _N, _FOUT), lambda g: (0, 0))],
        out_shape=[jax.ShapeDtypeStruct((_B * _N, _FOUT), jnp.float32),
                   jax.ShapeDtypeStruct((_B * _N, _FOUT), jnp.float32)],
    )(h2, W_l, W_r)
    gl = gl.reshape(_B, _N, _FOUT)
    gr = gr.reshape(_B, _N, _FOUT)

    nprog = _B * (_H // 2)
    e = pl.pallas_call(
        _score_kernel,
        grid=(_B, _H // 2),
        in_specs=[pl.BlockSpec((1, _N, 2 * _NH), lambda b, hp: (b, 0, hp)),
                  pl.BlockSpec((1, _N, 2 * _NH), lambda b, hp: (b, 0, hp)),
                  pl.BlockSpec((1, _N, _N), lambda b, hp: (b, 0, 0)),
                  pl.BlockSpec((1, _NH), lambda b, hp: (0, 0))],
        out_specs=pl.BlockSpec((1, 2, _N, _NP),
                               lambda b, hp: (b * (_H // 2) + hp, 0, 0, 0)),
        out_shape=jax.ShapeDtypeStruct((nprog, 2, _N, _NP), jnp.float32),
        compiler_params=pltpu.CompilerParams(
            dimension_semantics=("parallel", "parallel")),
    )(gl, gr, adj, attn_w)

    out = pl.pallas_call(
        _select_kernel,
        grid=(_B, _H // 2),
        in_specs=[pl.BlockSpec((1, 2, _N, _NP),
                               lambda b, hp: (b * (_H // 2) + hp, 0, 0, 0)),
                  pl.BlockSpec((1, _N, 2 * _NH), lambda b, hp: (b, 0, hp))],
        out_specs=pl.BlockSpec((1, 2, _N, _NH),
                               lambda b, hp: (b * (_H // 2) + hp, 0, 0, 0)),
        out_shape=jax.ShapeDtypeStruct((nprog, 2, _N, _NH), jnp.float32),
        compiler_params=pltpu.CompilerParams(
            dimension_semantics=("parallel", "parallel")),
    )(e, gr)

    out = out.reshape(_B, _H // 2, 2, _N, _NH)
    out = jnp.transpose(out, (0, 3, 1, 2, 4)).reshape(_B, _N, _FOUT)
    return out
